# baseline (speedup 1.0000x reference)
import jax
import jax.numpy as jnp
from jax.experimental import pallas as pl
from jax.experimental.pallas import tpu as pltpu

_IMG_W = 800
_IMG_H = 800
_OH = 7
_OW = 7
_H = 200
_W = 200
_C = 256
_TH = _H + 1

_RB = 8
_NC = _H // _RB
_BB = 32


def _roipool_kernel(
    idx_ref, inv_ref, fm_hbm, out_ref, tab_ref, acc_ref, tri_ref, fmbuf, sems
):
    @pl.when(pl.program_id(0) == 0)
    def _():
        acc_ref[...] = jnp.zeros_like(acc_ref)
        ii = jax.lax.broadcasted_iota(jnp.int32, (_TH, _W), 0)
        kk = jax.lax.broadcasted_iota(jnp.int32, (_TH, _W), 1)
        tri_ref[...] = jnp.where(kk < ii, 1.0, 0.0)

        pltpu.make_async_copy(
            fm_hbm.at[0, :, pl.ds(0, _RB), :], fmbuf.at[0], sems.at[0]
        ).start()

        def chunk(c, _):
            slot = c % 2
            pltpu.make_async_copy(
                fmbuf.at[slot], fmbuf.at[slot], sems.at[slot]
            ).wait()

            @pl.when(c + 1 < _NC)
            def _():
                nslot = (c + 1) % 2
                pltpu.make_async_copy(
                    fm_hbm.at[0, :, pl.ds((c + 1) * _RB, _RB), :],
                    fmbuf.at[nslot],
                    sems.at[nslot],
                ).start()

            base = c * _RB * _TH
            for k in range(_RB):
                tab_ref[pl.ds(base + k * _TH, _TH), 0, :] = acc_ref[...]
                x = fmbuf[slot, :, k, :]
                pref = jax.lax.dot_general(
                    tri_ref[...],
                    x,
                    (((1,), (1,)), ((), ())),
                    precision=jax.lax.Precision.HIGHEST,
                    preferred_element_type=jnp.float32,
                )
                acc_ref[...] = acc_ref[...] + pref
            return 0

        jax.lax.fori_loop(0, _NC, chunk, 0)
        tab_ref[pl.ds(_H * _TH, _TH), 0, :] = acc_ref[...]

    for bi in range(_BB):
        for i in range(_OH):
            vals = []
            for j in range(_OW):
                base = 4 * (i * _OW + j)
                vals.append(
                    (
                        tab_ref[idx_ref[bi, 0, base + 0], 0, :],
                        tab_ref[idx_ref[bi, 0, base + 1], 0, :],
                        tab_ref[idx_ref[bi, 0, base + 2], 0, :],
                        tab_ref[idx_ref[bi, 0, base + 3], 0, :],
                    )
                )
            for j in range(_OW):
                va, vb, vc, vd = vals[j]
                out_ref[bi, i, j, :] = (va - vb - vc + vd) * inv_ref[
                    bi, 0, i * _OW + j
                ]


def kernel(feature_map, boxes):
    n = boxes.shape[0]

    scale = jnp.array(
        [_W / _IMG_W, _H / _IMG_H, _W / _IMG_W, _H / _IMG_H], dtype=boxes.dtype
    )
    b = jnp.floor(boxes * scale).astype(jnp.int32)
    x1 = jnp.clip(b[:, 0], 0, _W - 1)
    y1 = jnp.clip(b[:, 1], 0, _H - 1)
    x2 = jnp.clip(b[:, 2] + 1, x1 + 1, _W)
    y2 = jnp.clip(b[:, 3] + 1, y1 + 1, _H)
    rh = y2 - y1
    rw = x2 - x1
    ii = jnp.arange(_OH)
    jj = jnp.arange(_OW)
    rs = y1[:, None] + (ii[None, :] * rh[:, None]) // _OH
    re = y1[:, None] + ((ii[None, :] + 1) * rh[:, None] + _OH - 1) // _OH
    cs = x1[:, None] + (jj[None, :] * rw[:, None]) // _OW
    ce = x1[:, None] + ((jj[None, :] + 1) * rw[:, None] + _OW - 1) // _OW

    f_a = re[:, :, None] * _TH + ce[:, None, :]
    f_b = rs[:, :, None] * _TH + ce[:, None, :]
    f_c = re[:, :, None] * _TH + cs[:, None, :]
    f_d = rs[:, :, None] * _TH + cs[:, None, :]
    idx = (
        jnp.stack([f_a, f_b, f_c, f_d], axis=-1)
        .reshape(n, 1, 4 * _OH * _OW)
        .astype(jnp.int32)
    )
    area = ((re - rs)[:, :, None] * (ce - cs)[:, None, :]).astype(jnp.float32)
    inv = (1.0 / area).reshape(n, 1, _OH * _OW)

    out = pl.pallas_call(
        _roipool_kernel,
        grid=(n // _BB,),
        in_specs=[
            pl.BlockSpec(
                (_BB, 1, 4 * _OH * _OW),
                lambda s: (s, 0, 0),
                memory_space=pltpu.SMEM,
            ),
            pl.BlockSpec(
                (_BB, 1, _OH * _OW),
                lambda s: (s, 0, 0),
                memory_space=pltpu.SMEM,
            ),
            pl.BlockSpec(memory_space=pl.ANY),
        ],
        out_specs=pl.BlockSpec((_BB, _OH, _OW, _C), lambda s: (s, 0, 0, 0)),
        out_shape=jax.ShapeDtypeStruct((n, _OH, _OW, _C), jnp.float32),
        scratch_shapes=[
            pltpu.VMEM((_TH * _TH, 1, _C), jnp.float32),
            pltpu.VMEM((_TH, _C), jnp.float32),
            pltpu.VMEM((_TH, _W), jnp.float32),
            pltpu.VMEM((2, _C, _RB, _W), jnp.float32),
            pltpu.SemaphoreType.DMA((2,)),
        ],
        compiler_params=pltpu.CompilerParams(
            dimension_semantics=("arbitrary",),
            vmem_limit_bytes=56 * 1024 * 1024,
        ),
    )(idx, inv, feature_map)

    return out.transpose(0, 3, 1, 2)


# speedup vs baseline: 1.0041x; 1.0041x over previous
import jax
import jax.numpy as jnp
from jax.experimental import pallas as pl
from jax.experimental.pallas import tpu as pltpu

_IMG_W = 800
_IMG_H = 800
_OH = 7
_OW = 7
_H = 200
_W = 200
_C = 256
_TH = _H + 1

_RB = 8
_NC = _H // _RB
_BB = 64


def _roipool_kernel(
    idx_ref, fm_hbm, out_ref, tab_ref, acc_ref, tri_ref, fmbuf, sems
):
    @pl.when(pl.program_id(0) == 0)
    def _():
        acc_ref[...] = jnp.zeros_like(acc_ref)
        ii = jax.lax.broadcasted_iota(jnp.int32, (_TH, _W), 0)
        kk = jax.lax.broadcasted_iota(jnp.int32, (_TH, _W), 1)
        tri_ref[...] = jnp.where(kk < ii, 1.0, 0.0)

        pltpu.make_async_copy(
            fm_hbm.at[0, :, pl.ds(0, _RB), :], fmbuf.at[0], sems.at[0]
        ).start()

        def chunk(c, _):
            slot = c % 2
            pltpu.make_async_copy(
                fmbuf.at[slot], fmbuf.at[slot], sems.at[slot]
            ).wait()

            @pl.when(c + 1 < _NC)
            def _():
                nslot = (c + 1) % 2
                pltpu.make_async_copy(
                    fm_hbm.at[0, :, pl.ds((c + 1) * _RB, _RB), :],
                    fmbuf.at[nslot],
                    sems.at[nslot],
                ).start()

            base = c * _RB * _TH
            for k in range(_RB):
                tab_ref[pl.ds(base + k * _TH, _TH), 0, :] = acc_ref[...]
                x = fmbuf[slot, :, k, :]
                pref = jax.lax.dot_general(
                    tri_ref[...],
                    x,
                    (((1,), (1,)), ((), ())),
                    precision=jax.lax.Precision.HIGHEST,
                    preferred_element_type=jnp.float32,
                )
                acc_ref[...] = acc_ref[...] + pref
            return 0

        jax.lax.fori_loop(0, _NC, chunk, 0)
        tab_ref[pl.ds(_H * _TH, _TH), 0, :] = acc_ref[...]

    for bi in range(_BB):
        for i in range(_OH):
            vals = []
            for j in range(_OW):
                base = 4 * (i * _OW + j)
                vals.append(
                    (
                        tab_ref[idx_ref[bi, 0, base + 0], 0, :],
                        tab_ref[idx_ref[bi, 0, base + 1], 0, :],
                        tab_ref[idx_ref[bi, 0, base + 2], 0, :],
                        tab_ref[idx_ref[bi, 0, base + 3], 0, :],
                    )
                )
            for j in range(_OW):
                va, vb, vc, vd = vals[j]
                out_ref[bi, i, j, :] = va - vb - vc + vd


def kernel(feature_map, boxes):
    n = boxes.shape[0]

    scale = jnp.array(
        [_W / _IMG_W, _H / _IMG_H, _W / _IMG_W, _H / _IMG_H], dtype=boxes.dtype
    )
    b = jnp.floor(boxes * scale).astype(jnp.int32)
    x1 = jnp.clip(b[:, 0], 0, _W - 1)
    y1 = jnp.clip(b[:, 1], 0, _H - 1)
    x2 = jnp.clip(b[:, 2] + 1, x1 + 1, _W)
    y2 = jnp.clip(b[:, 3] + 1, y1 + 1, _H)
    rh = y2 - y1
    rw = x2 - x1
    ii = jnp.arange(_OH)
    jj = jnp.arange(_OW)
    rs = y1[:, None] + (ii[None, :] * rh[:, None]) // _OH
    re = y1[:, None] + ((ii[None, :] + 1) * rh[:, None] + _OH - 1) // _OH
    cs = x1[:, None] + (jj[None, :] * rw[:, None]) // _OW
    ce = x1[:, None] + ((jj[None, :] + 1) * rw[:, None] + _OW - 1) // _OW

    f_a = re[:, :, None] * _TH + ce[:, None, :]
    f_b = rs[:, :, None] * _TH + ce[:, None, :]
    f_c = re[:, :, None] * _TH + cs[:, None, :]
    f_d = rs[:, :, None] * _TH + cs[:, None, :]
    idx = (
        jnp.stack([f_a, f_b, f_c, f_d], axis=-1)
        .reshape(n, 1, 4 * _OH * _OW)
        .astype(jnp.int32)
    )
    area = ((re - rs)[:, :, None] * (ce - cs)[:, None, :]).astype(jnp.float32)

    out = pl.pallas_call(
        _roipool_kernel,
        grid=(n // _BB,),
        in_specs=[
            pl.BlockSpec(
                (_BB, 1, 4 * _OH * _OW),
                lambda s: (s, 0, 0),
                memory_space=pltpu.SMEM,
            ),
            pl.BlockSpec(memory_space=pl.ANY),
        ],
        out_specs=pl.BlockSpec((_BB, _OH, _OW, _C), lambda s: (s, 0, 0, 0)),
        out_shape=jax.ShapeDtypeStruct((n, _OH, _OW, _C), jnp.float32),
        scratch_shapes=[
            pltpu.VMEM((_TH * _TH, 1, _C), jnp.float32),
            pltpu.VMEM((_TH, _C), jnp.float32),
            pltpu.VMEM((_TH, _W), jnp.float32),
            pltpu.VMEM((2, _C, _RB, _W), jnp.float32),
            pltpu.SemaphoreType.DMA((2,)),
        ],
        compiler_params=pltpu.CompilerParams(
            dimension_semantics=("arbitrary",),
            vmem_limit_bytes=56 * 1024 * 1024,
        ),
    )(idx, feature_map)

    return out.transpose(0, 3, 1, 2) / area[:, None, :, :]


# speedup vs baseline: 1.0124x; 1.0083x over previous
import jax
import jax.numpy as jnp
from jax.experimental import pallas as pl
from jax.experimental.pallas import tpu as pltpu

_IMG_W = 800
_IMG_H = 800
_OH = 7
_OW = 7
_H = 200
_W = 200
_C = 256
_TH = _H + 1

_RB = 8
_NC = _H // _RB
_BB = 64
_KS = 128


def _roipool_kernel(
    idx_ref, fm_hbm, out_ref, tab_ref, acc_ref, tri_a_ref, tri_b_ref, fmbuf, sems
):
    @pl.when(pl.program_id(0) == 0)
    def _():
        acc_ref[...] = jnp.zeros_like(acc_ref)
        ia = jax.lax.broadcasted_iota(jnp.int32, (_KS + 1, _KS), 0)
        ka = jax.lax.broadcasted_iota(jnp.int32, (_KS + 1, _KS), 1)
        tri_a_ref[...] = jnp.where(ka < ia, 1.0, 0.0)
        ib = jax.lax.broadcasted_iota(jnp.int32, (_TH - _KS, _W - _KS), 0)
        kb = jax.lax.broadcasted_iota(jnp.int32, (_TH - _KS, _W - _KS), 1)
        tri_b_ref[...] = jnp.where(kb < ib, 1.0, 0.0)

        pltpu.make_async_copy(
            fm_hbm.at[0, :, pl.ds(0, _RB), :], fmbuf.at[0], sems.at[0]
        ).start()

        def chunk(c, _):
            slot = c % 2
            pltpu.make_async_copy(
                fmbuf.at[slot], fmbuf.at[slot], sems.at[slot]
            ).wait()

            @pl.when(c + 1 < _NC)
            def _():
                nslot = (c + 1) % 2
                pltpu.make_async_copy(
                    fm_hbm.at[0, :, pl.ds((c + 1) * _RB, _RB), :],
                    fmbuf.at[nslot],
                    sems.at[nslot],
                ).start()

            base = c * _RB * _TH
            for k in range(_RB):
                tab_ref[pl.ds(base + k * _TH, _TH), 0, :] = acc_ref[...]
                dims = (((1,), (1,)), ((), ()))
                xa = fmbuf[slot, :, k, 0:_KS]
                xb = fmbuf[slot, :, k, _KS:_W]
                pt = jax.lax.dot_general(
                    tri_a_ref[...],
                    xa,
                    dims,
                    precision=jax.lax.Precision.HIGHEST,
                    preferred_element_type=jnp.float32,
                )
                pb = jax.lax.dot_general(
                    tri_b_ref[...],
                    xb,
                    dims,
                    precision=jax.lax.Precision.HIGHEST,
                    preferred_element_type=jnp.float32,
                )
                pref = jnp.concatenate(
                    [pt[0:_KS], pt[_KS : _KS + 1] + pb], axis=0
                )
                acc_ref[...] = acc_ref[...] + pref
            return 0

        jax.lax.fori_loop(0, _NC, chunk, 0)
        tab_ref[pl.ds(_H * _TH, _TH), 0, :] = acc_ref[...]

    for bi in range(_BB):
        for i in range(_OH):
            vals = []
            for j in range(_OW):
                base = 4 * (i * _OW + j)
                vals.append(
                    (
                        tab_ref[idx_ref[bi, 0, base + 0], 0, :],
                        tab_ref[idx_ref[bi, 0, base + 1], 0, :],
                        tab_ref[idx_ref[bi, 0, base + 2], 0, :],
                        tab_ref[idx_ref[bi, 0, base + 3], 0, :],
                    )
                )
            for j in range(_OW):
                va, vb, vc, vd = vals[j]
                out_ref[bi, i, j, :] = va - vb - vc + vd


def kernel(feature_map, boxes):
    n = boxes.shape[0]

    scale = jnp.array(
        [_W / _IMG_W, _H / _IMG_H, _W / _IMG_W, _H / _IMG_H], dtype=boxes.dtype
    )
    b = jnp.floor(boxes * scale).astype(jnp.int32)
    x1 = jnp.clip(b[:, 0], 0, _W - 1)
    y1 = jnp.clip(b[:, 1], 0, _H - 1)
    x2 = jnp.clip(b[:, 2] + 1, x1 + 1, _W)
    y2 = jnp.clip(b[:, 3] + 1, y1 + 1, _H)
    rh = y2 - y1
    rw = x2 - x1
    ii = jnp.arange(_OH)
    jj = jnp.arange(_OW)
    rs = y1[:, None] + (ii[None, :] * rh[:, None]) // _OH
    re = y1[:, None] + ((ii[None, :] + 1) * rh[:, None] + _OH - 1) // _OH
    cs = x1[:, None] + (jj[None, :] * rw[:, None]) // _OW
    ce = x1[:, None] + ((jj[None, :] + 1) * rw[:, None] + _OW - 1) // _OW

    f_a = re[:, :, None] * _TH + ce[:, None, :]
    f_b = rs[:, :, None] * _TH + ce[:, None, :]
    f_c = re[:, :, None] * _TH + cs[:, None, :]
    f_d = rs[:, :, None] * _TH + cs[:, None, :]
    idx = (
        jnp.stack([f_a, f_b, f_c, f_d], axis=-1)
        .reshape(n, 1, 4 * _OH * _OW)
        .astype(jnp.int32)
    )
    area = ((re - rs)[:, :, None] * (ce - cs)[:, None, :]).astype(jnp.float32)

    out = pl.pallas_call(
        _roipool_kernel,
        grid=(n // _BB,),
        in_specs=[
            pl.BlockSpec(
                (_BB, 1, 4 * _OH * _OW),
                lambda s: (s, 0, 0),
                memory_space=pltpu.SMEM,
            ),
            pl.BlockSpec(memory_space=pl.ANY),
        ],
        out_specs=pl.BlockSpec((_BB, _OH, _OW, _C), lambda s: (s, 0, 0, 0)),
        out_shape=jax.ShapeDtypeStruct((n, _OH, _OW, _C), jnp.float32),
        scratch_shapes=[
            pltpu.VMEM((_TH * _TH, 1, _C), jnp.float32),
            pltpu.VMEM((_TH, _C), jnp.float32),
            pltpu.VMEM((_KS + 1, _KS), jnp.float32),
            pltpu.VMEM((_TH - _KS, _W - _KS), jnp.float32),
            pltpu.VMEM((2, _C, _RB, _W), jnp.float32),
            pltpu.SemaphoreType.DMA((2,)),
        ],
        compiler_params=pltpu.CompilerParams(
            dimension_semantics=("arbitrary",),
            vmem_limit_bytes=56 * 1024 * 1024,
        ),
    )(idx, feature_map)

    return out.transpose(0, 3, 1, 2) / area[:, None, :, :]
